# baseline (speedup 1.0000x reference)
import numpy as np

import jax
import jax.numpy as jnp
from jax.experimental import pallas as pl
from jax.experimental.pallas import tpu as pltpu


def _enc_ind(in_size, out_size):
    R = np.zeros((3, in_size, out_size), np.float32)
    for kw in range(3):
        for s in range(out_size):
            j = 2 * s + kw - 1
            if 0 <= j < in_size:
                R[kw, j, s] = 1.0
    return R


def _dec_ind(in_size, out_size):
    R = np.zeros((3, in_size, out_size), np.float32)
    for kw in range(3):
        for s in range(out_size):
            v = s + kw - 1
            if v % 2 == 0 and 0 <= v // 2 < in_size:
                R[kw, v // 2, s] = 1.0
    return R


_RE1 = _enc_ind(28, 14)
_RE2 = _enc_ind(14, 7)
_RD2 = _dec_ind(7, 14)
_RD3 = _dec_ind(14, 28)

def _enc_pairs(in_size, out_size):
    return [(kh, 2 * r + kh - 1, r) for r in range(out_size) for kh in range(3)
            if 0 <= 2 * r + kh - 1 < in_size]


def _dec_pairs(in_size, out_size):
    out = []
    for r in range(out_size):
        for kh in range(3):
            v = r + kh - 1
            if v % 2 == 0 and 0 <= v // 2 < in_size:
                out.append((kh, v // 2, r))
    return out


_PE1 = _enc_pairs(28, 14)
_PE2 = _enc_pairs(14, 7)
_PD2 = _dec_pairs(7, 14)
_PD3 = _dec_pairs(14, 28)


def _taps(w_packed, cout, cin, R):
    w9 = w_packed.reshape(cout, 3, 3, cin)
    T = jnp.einsum('ohwc,wjs->hcjos', w9, R)
    return T.reshape(3, cin * R.shape[1], cout * R.shape[2])


def _interleave(evens, odds, bsz):
    ne = evens.shape[0] // bsz
    pieces = []
    for k in range(ne):
        pieces.append(evens[k * bsz:(k + 1) * bsz])
        if (k + 1) * bsz <= odds.shape[0]:
            pieces.append(odds[k * bsz:(k + 1) * bsz])
    return jnp.concatenate(pieces, axis=0)


def _fused_kernel(x_ref, te1_ref, be1_ref, te2_ref, be2_ref, w3_ref, b3_ref,
                  w4_ref, b4_ref, td2_ref, bd2_ref, td3_ref, bd3_ref,
                  enc_ref, dec_ref):
    f32 = jnp.float32
    bsz = x_ref.shape[0]

    def dot(a, w):
        return jnp.dot(a, w, preferred_element_type=f32)

    def xrow(i):
        return jnp.concatenate([x_ref[:, c, i, :] for c in range(3)],
                               axis=1)
    x_even = jnp.concatenate([xrow(i) for i in range(0, 28, 2)], axis=0)
    x_odd = jnp.concatenate([xrow(i) for i in range(1, 28, 2)], axis=0)

    y0 = dot(x_odd[:13 * bsz], te1_ref[0])
    y1 = dot(x_even, te1_ref[1])
    y2 = dot(x_odd, te1_ref[2])
    z = y1 + y2
    z = jnp.concatenate([z[:bsz], z[bsz:] + y0], axis=0)
    x2 = jnp.maximum(z + be1_ref[...], 0.0)

    x2_even = jnp.concatenate([x2[i * bsz:(i + 1) * bsz]
                               for i in range(0, 14, 2)], axis=0)
    x2_odd = jnp.concatenate([x2[i * bsz:(i + 1) * bsz]
                              for i in range(1, 14, 2)], axis=0)
    y0 = dot(x2_odd[:6 * bsz], te2_ref[0])
    y1 = dot(x2_even, te2_ref[1])
    y2 = dot(x2_odd, te2_ref[2])
    z = y1 + y2
    z = jnp.concatenate([z[:bsz], z[bsz:] + y0], axis=0)
    z2s = jnp.maximum(z + be2_ref[...], 0.0)

    z2 = jnp.concatenate([z2s[r * bsz:(r + 1) * bsz] for r in range(7)],
                         axis=1)
    enc = dot(z2, w3_ref[...]) + b3_ref[...]
    enc_ref[...] = enc

    h = jnp.maximum(dot(enc, w4_ref[...]) + b4_ref[...], 0.0)
    hs = jnp.concatenate([h[:, i * 224:(i + 1) * 224] for i in range(7)],
                         axis=0)

    y0 = dot(hs, td2_ref[0])
    y1 = dot(hs, td2_ref[1])
    y2 = dot(hs[bsz:], td2_ref[2])
    zodd = jnp.concatenate([y0[:6 * bsz] + y2, y0[6 * bsz:]], axis=0)
    zd2 = jnp.maximum(_interleave(y1, zodd, bsz) + bd2_ref[...], 0.0)

    y0 = dot(zd2, td3_ref[0])
    y1 = dot(zd2, td3_ref[1])
    y2 = dot(zd2[bsz:], td3_ref[2])
    zodd = jnp.concatenate([y0[:13 * bsz] + y2, y0[13 * bsz:]], axis=0)
    a3 = jnp.tanh(_interleave(y1, zodd, bsz) + bd3_ref[...])
    for r in range(28):
        for o in range(3):
            dec_ref[:, o, r, :] = a3[r * bsz:(r + 1) * bsz, o * 28:(o + 1) * 28]


def _const_spec(shape):
    return pl.BlockSpec(shape, lambda i: tuple(0 for _ in shape))


def kernel(x_nchw, w1, b1, w2, b2, we3, be3, wd1, bd1,
           wd2, bd2, wd3, bd3, s1, sd2, sd3):
    f32 = jnp.float32
    n = x_nchw.shape[0]
    bm = 128 if n % 128 == 0 else n
    nb = n // bm

    TE1 = _taps(w1, 16, 3, _RE1)
    BE1 = jnp.repeat(b1.reshape(-1), 14).reshape(1, 224)
    TE2 = _taps(w2, 32, 16, _RE2)
    BE2 = jnp.repeat(b2.reshape(-1), 7).reshape(1, 224)
    W3 = jnp.transpose(we3.reshape(7, 7, 32, 64), (0, 2, 1, 3)).reshape(1568, 64)
    B3 = be3.reshape(1, 64)
    W4 = jnp.transpose(wd1.reshape(64, 32, 7, 7), (0, 2, 1, 3)).reshape(64, 1568)
    B4 = jnp.transpose(bd1.reshape(32, 7, 7), (1, 0, 2)).reshape(1, 1568)
    TD2 = _taps(wd2, 16, 32, _RD2)
    BD2 = jnp.repeat(bd2.reshape(-1), 14).reshape(1, 224)
    TD3 = _taps(wd3, 3, 16, _RD3)
    BD3 = jnp.repeat(bd3.reshape(-1), 28).reshape(1, 84)

    enc, dec = pl.pallas_call(
        _fused_kernel,
        out_shape=(jax.ShapeDtypeStruct((n, 64), f32),
                   jax.ShapeDtypeStruct((n, 3, 28, 28), f32)),
        grid=(nb,),
        in_specs=[pl.BlockSpec((bm, 3, 28, 28), lambda i: (i, 0, 0, 0)),
                  _const_spec((3, 84, 224)), _const_spec((1, 224)),
                  _const_spec((3, 224, 224)), _const_spec((1, 224)),
                  _const_spec((1568, 64)), _const_spec((1, 64)),
                  _const_spec((64, 1568)), _const_spec((1, 1568)),
                  _const_spec((3, 224, 224)), _const_spec((1, 224)),
                  _const_spec((3, 224, 84)), _const_spec((1, 84))],
        out_specs=(pl.BlockSpec((bm, 64), lambda i: (i, 0)),
                   pl.BlockSpec((bm, 3, 28, 28), lambda i: (i, 0, 0, 0))),
        compiler_params=pltpu.CompilerParams(dimension_semantics=("parallel",)),
        cost_estimate=pl.CostEstimate(flops=2 * n * 7_500_000,
                                      transcendentals=n * 2352,
                                      bytes_accessed=4 * (n * 2352 + n * 2352 + n * 64)),
    )(x_nchw, TE1, BE1, TE2, BE2, W3, B3, W4, B4, TD2, BD2, TD3, BD3)

    return enc.reshape(n, 64, 1, 1), dec


# speedup vs baseline: 2.3593x; 2.3593x over previous
import numpy as np

import jax
import jax.numpy as jnp
from jax.experimental import pallas as pl
from jax.experimental.pallas import tpu as pltpu


def _enc_ind(in_size, out_size):
    R = np.zeros((3, in_size, out_size), np.float32)
    for kw in range(3):
        for s in range(out_size):
            j = 2 * s + kw - 1
            if 0 <= j < in_size:
                R[kw, j, s] = 1.0
    return R


def _dec_ind(in_size, out_size):
    R = np.zeros((3, in_size, out_size), np.float32)
    for kw in range(3):
        for s in range(out_size):
            v = s + kw - 1
            if v % 2 == 0 and 0 <= v // 2 < in_size:
                R[kw, v // 2, s] = 1.0
    return R


_RE1 = _enc_ind(28, 14)
_RE2 = _enc_ind(14, 7)
_RD2 = _dec_ind(7, 14)
_RD3 = _dec_ind(14, 28)

def _enc_pairs(in_size, out_size):
    return [(kh, 2 * r + kh - 1, r) for r in range(out_size) for kh in range(3)
            if 0 <= 2 * r + kh - 1 < in_size]


def _dec_pairs(in_size, out_size):
    out = []
    for r in range(out_size):
        for kh in range(3):
            v = r + kh - 1
            if v % 2 == 0 and 0 <= v // 2 < in_size:
                out.append((kh, v // 2, r))
    return out


_PE1 = _enc_pairs(28, 14)
_PE2 = _enc_pairs(14, 7)
_PD2 = _dec_pairs(7, 14)
_PD3 = _dec_pairs(14, 28)


def _taps(w_packed, cout, cin, R):
    w9 = w_packed.reshape(cout, 3, 3, cin)
    T = jnp.einsum('ohwc,wjs->hcjos', w9, R)
    return T.reshape(3, cin * R.shape[1], cout * R.shape[2])


def _fused_kernel(x_ref, te1_ref, be1_ref, te2_ref, be2_ref, w3_ref, b3_ref,
                  w4_ref, b4_ref, td2_ref, bd2_ref, td3_ref, bd3_ref,
                  enc_ref, dec_ref):
    f32 = jnp.float32
    x = x_ref[...]

    xrows = [jnp.concatenate([x[:, c * 784 + i * 28:c * 784 + i * 28 + 28]
                              for c in range(3)], axis=1) for i in range(28)]

    acc = {}
    for kh, i, r in _PE1:
        t = jnp.dot(xrows[i], te1_ref[kh], preferred_element_type=f32)
        acc[r] = t if r not in acc else acc[r] + t
    x2 = [jnp.maximum(acc[r] + be1_ref[...], 0.0) for r in range(14)]

    acc = {}
    for kh, i, r in _PE2:
        t = jnp.dot(x2[i], te2_ref[kh], preferred_element_type=f32)
        acc[r] = t if r not in acc else acc[r] + t
    z2 = jnp.concatenate([jnp.maximum(acc[r] + be2_ref[...], 0.0)
                          for r in range(7)], axis=1)

    enc = jnp.dot(z2, w3_ref[...], preferred_element_type=f32) + b3_ref[...]
    enc_ref[...] = enc

    h = jnp.maximum(jnp.dot(enc, w4_ref[...], preferred_element_type=f32)
                    + b4_ref[...], 0.0)
    hrows = [h[:, i * 224:(i + 1) * 224] for i in range(7)]

    acc = {}
    for kh, i, r in _PD2:
        t = jnp.dot(hrows[i], td2_ref[kh], preferred_element_type=f32)
        acc[r] = t if r not in acc else acc[r] + t
    zd2 = [jnp.maximum(acc[r] + bd2_ref[...], 0.0) for r in range(14)]

    acc = {}
    for kh, i, r in _PD3:
        t = jnp.dot(zd2[i], td3_ref[kh], preferred_element_type=f32)
        acc[r] = t if r not in acc else acc[r] + t
    for r in range(28):
        a3 = jnp.tanh(acc[r] + bd3_ref[...])
        for o in range(3):
            dec_ref[:, o * 784 + r * 28:o * 784 + r * 28 + 28] = \
                a3[:, o * 28:(o + 1) * 28]


def _const_spec(shape):
    return pl.BlockSpec(shape, lambda i: tuple(0 for _ in shape))


def kernel(x_nchw, w1, b1, w2, b2, we3, be3, wd1, bd1,
           wd2, bd2, wd3, bd3, s1, sd2, sd3):
    f32 = jnp.float32
    n = x_nchw.shape[0]
    bm = 512 if n % 512 == 0 else n
    nb = n // bm

    x2d = x_nchw.reshape(n, 3 * 784)

    TE1 = _taps(w1, 16, 3, _RE1)
    BE1 = jnp.repeat(b1.reshape(-1), 14).reshape(1, 224)
    TE2 = _taps(w2, 32, 16, _RE2)
    BE2 = jnp.repeat(b2.reshape(-1), 7).reshape(1, 224)
    W3 = jnp.transpose(we3.reshape(7, 7, 32, 64), (0, 2, 1, 3)).reshape(1568, 64)
    B3 = be3.reshape(1, 64)
    W4 = jnp.transpose(wd1.reshape(64, 32, 7, 7), (0, 2, 1, 3)).reshape(64, 1568)
    B4 = jnp.transpose(bd1.reshape(32, 7, 7), (1, 0, 2)).reshape(1, 1568)
    TD2 = _taps(wd2, 16, 32, _RD2)
    BD2 = jnp.repeat(bd2.reshape(-1), 14).reshape(1, 224)
    TD3 = _taps(wd3, 3, 16, _RD3)
    BD3 = jnp.repeat(bd3.reshape(-1), 28).reshape(1, 84)

    enc, dec = pl.pallas_call(
        _fused_kernel,
        out_shape=(jax.ShapeDtypeStruct((n, 64), f32),
                   jax.ShapeDtypeStruct((n, 2352), f32)),
        grid=(nb,),
        in_specs=[pl.BlockSpec((bm, 2352), lambda i: (i, 0)),
                  _const_spec((3, 84, 224)), _const_spec((1, 224)),
                  _const_spec((3, 224, 224)), _const_spec((1, 224)),
                  _const_spec((1568, 64)), _const_spec((1, 64)),
                  _const_spec((64, 1568)), _const_spec((1, 1568)),
                  _const_spec((3, 224, 224)), _const_spec((1, 224)),
                  _const_spec((3, 224, 84)), _const_spec((1, 84))],
        out_specs=(pl.BlockSpec((bm, 64), lambda i: (i, 0)),
                   pl.BlockSpec((bm, 2352), lambda i: (i, 0))),
        compiler_params=pltpu.CompilerParams(dimension_semantics=("parallel",)),
        cost_estimate=pl.CostEstimate(flops=2 * n * 7_500_000,
                                      transcendentals=n * 2352,
                                      bytes_accessed=4 * (n * 2352 + n * 2352 + n * 64)),
    )(x2d, TE1, BE1, TE2, BE2, W3, B3, W4, B4, TD2, BD2, TD3, BD3)

    return enc.reshape(n, 64, 1, 1), dec.reshape(n, 3, 28, 28)
